# Initial kernel scaffold; baseline (speedup 1.0000x reference)
import jax
import jax.numpy as jnp
from jax.experimental import pallas as pl


def kernel(positions, embeddings, cell, species):
    raise NotImplementedError("write your pallas kernel here")


# speedup vs baseline: 1.0001x; 1.0001x over previous
import jax
import jax.numpy as jnp
from jax.experimental import pallas as pl


def kernel(positions, embeddings, cell, species):
    MESH_RESOLUTION = 0.1
    BOX = 12.8
    box_size = jnp.trace(cell) / 3.0
    n_mesh = 2 * int(round(BOX / (2 * MESH_RESOLUTION)))
    spacing = box_size / n_mesh
    n_channels = embeddings.shape[1]

    positions_cell = positions / spacing
    positions_cell_idx = jnp.round(positions_cell).astype(jnp.int32)
    l_dist = positions_cell - positions_cell_idx
    r_dist = 1.0 - l_dist

    ix = positions_cell_idx[:, 0]
    iy = positions_cell_idx[:, 1]
    iz = positions_cell_idx[:, 2]
    eT = embeddings.T

    w = jnp.zeros((n_channels, n_mesh, n_mesh, n_mesh), dtype=positions.dtype)
    for dx, wx in ((0, l_dist[:, 0]), (1, r_dist[:, 0])):
        for dy, wy in ((0, l_dist[:, 1]), (1, r_dist[:, 1])):
            for dz, wz in ((0, l_dist[:, 2]), (1, r_dist[:, 2])):
                frac = wx * wy * wz
                w = w.at[:, (ix + dx) % n_mesh, (iy + dy) % n_mesh, (iz + dz) % n_mesh].add(frac * eT)
    return w


# speedup vs baseline: 1.2984x; 1.2983x over previous
import functools

import jax
import jax.numpy as jnp
from jax import lax
from jax.experimental import pallas as pl
from jax.experimental.pallas import tpu as pltpu
from jax.experimental.pallas import tpu_sc as plsc

N_MESH = 128
N_CH = 16
N_ATOMS_PAD = 100352
SHARE = N_ATOMS_PAD // 16
CHUNKS = SHARE // 16
XW = 2
N_PASS = N_MESH // (2 * XW)
PLANE = N_MESH * N_MESH
WROWS = XW * PLANE
TSH = WROWS // 16
ZROWS = 1024
RC = float(2 ** 23)

_GDN = lax.GatherDimensionNumbers(
    offset_dims=(), collapsed_slice_dims=(0,), start_index_map=(0,))


def _permute(v, idx):
  return lax.gather(v, idx[:, None], dimension_numbers=_GDN,
                    slice_sizes=(1,),
                    mode=lax.GatherScatterMode.PROMISE_IN_BOUNDS)


def _bcast_lane(v, a):
  return _permute(v, jnp.full((16,), a, jnp.int32))


_LANE16 = None


def _prefix_sum(x):
  lane = lax.iota(jnp.int32, 16)
  for k in (1, 2, 4, 8):
    sh = _permute(x, jnp.maximum(lane - k, 0))
    x = x + jnp.where(lane >= k, sh, 0)
  return x


@functools.partial(
    pl.kernel,
    mesh=plsc.VectorSubcoreMesh(core_axis_name="c", subcore_axis_name="s"),
    out_type=jax.ShapeDtypeStruct((N_MESH ** 3, N_CH), jnp.float32),
    compiler_params=pltpu.CompilerParams(use_tc_tiling_on_sc=False,
                                         needs_layout_passes=False),
    scratch_types=[
        pltpu.VMEM_SHARED((WROWS, N_CH), jnp.float32),
        pltpu.VMEM((SHARE,), jnp.float32),
        pltpu.VMEM((SHARE,), jnp.float32),
        pltpu.VMEM((SHARE,), jnp.float32),
        pltpu.VMEM((16,), jnp.float32),
        pltpu.VMEM((SHARE + 16,), jnp.int32),
        pltpu.VMEM((SHARE + 16,), jnp.int32),
        pltpu.VMEM((SHARE + 16,), jnp.float32),
        pltpu.VMEM((SHARE + 16,), jnp.float32),
        pltpu.VMEM((SHARE + 16,), jnp.float32),
        pltpu.VMEM((SHARE + 16,), jnp.float32),
        pltpu.VMEM((16,), jnp.int32),
        pltpu.VMEM((16, N_CH), jnp.float32),
        pltpu.VMEM((128, N_CH), jnp.float32),
        pltpu.VMEM((128,), jnp.int32),
        pltpu.VMEM((ZROWS, N_CH), jnp.float32),
        pltpu.SemaphoreType.DMA,
    ],
)
def _deposit(px_hbm, py_hbm, pz_hbm, emb_hbm, sp_hbm, out_hbm,
             window, pxv, pyv, pzv, spv,
             p_id, p_pk, p_wx0, p_wx1, p_ly, p_lz,
             gidx, emb_t, stage, idx_buf, zbuf, sem):
  c = lax.axis_index("c")
  s = lax.axis_index("s")
  a0 = s * SHARE

  pltpu.sync_copy(px_hbm.at[pl.ds(a0, SHARE)], pxv)
  pltpu.sync_copy(py_hbm.at[pl.ds(a0, SHARE)], pyv)
  pltpu.sync_copy(pz_hbm.at[pl.ds(a0, SHARE)], pzv)
  pltpu.sync_copy(sp_hbm, spv)

  def zrow(i, carry):
    zbuf[i] = jnp.zeros((N_CH,), jnp.float32)
    return carry
  lax.fori_loop(0, ZROWS, zrow, 0)

  spacing = spv[...]
  lane = lax.iota(jnp.int32, 16)

  def one_pass(p, carry):
    x0 = (2 * p + c) * XW

    for k in range(TSH // ZROWS):
      pltpu.sync_copy(zbuf, window.at[pl.ds(s * TSH + k * ZROWS, ZROWS)])
    plsc.subcore_barrier()

    def scan_chunk(i, cnt):
      b = i * 16
      pcx = pxv[pl.ds(b, 16)] / spacing
      pcy = pyv[pl.ds(b, 16)] / spacing
      pcz = pzv[pl.ds(b, 16)] / spacing
      fx = (pcx + RC) - RC
      fy = (pcy + RC) - RC
      fz = (pcz + RC) - RC
      ix = fx.astype(jnp.int32) & (N_MESH - 1)
      iy = fy.astype(jnp.int32) & (N_MESH - 1)
      iz = fz.astype(jnp.int32) & (N_MESH - 1)
      lx = pcx - fx
      ly = pcy - fy
      lz = pcz - fz
      d = (ix - x0 + 1) & (N_MESH - 1)
      match = d <= XW
      wx0 = jnp.where((d >= 1) & (d <= XW), lx, 0.0)
      wx1 = jnp.where(d <= XW - 1, 1.0 - lx, 0.0)
      rel0 = jnp.clip(d - 1, 0, XW - 1)
      rel1 = jnp.clip(d, 0, XW - 1)
      packed = rel0 | (rel1 << 2) | (iy << 4) | (iz << 11)
      gid = a0 + b + lane
      incl = _prefix_sum(match.astype(jnp.int32))
      pos = jnp.where(match, jnp.maximum(cnt + incl - 1, 0), SHARE)
      plsc.store_scatter(p_id, [pos], gid)
      plsc.store_scatter(p_pk, [pos], packed)
      plsc.store_scatter(p_wx0, [pos], wx0)
      plsc.store_scatter(p_wx1, [pos], wx1)
      plsc.store_scatter(p_ly, [pos], ly)
      plsc.store_scatter(p_lz, [pos], lz)
      return cnt + _bcast_lane(incl, 15)

    cnt = lax.fori_loop(0, CHUNKS, scan_chunk,
                        jnp.zeros((16,), jnp.int32))

    def dep_group(g):
      b = g * 16
      valid = (b + lane) < cnt
      ids = jnp.where(valid, p_id[pl.ds(b, 16)], 0)
      pk = p_pk[pl.ds(b, 16)]
      wxs = (jnp.where(valid, p_wx0[pl.ds(b, 16)], 0.0),
             jnp.where(valid, p_wx1[pl.ds(b, 16)], 0.0))
      lyv = p_ly[pl.ds(b, 16)]
      lzv = p_lz[pl.ds(b, 16)]
      gidx[...] = ids
      pltpu.async_copy(emb_hbm.at[gidx], emb_t, sem).wait()
      y0 = (pk >> 4) & (N_MESH - 1)
      z0 = (pk >> 11) & (N_MESH - 1)
      wys = (lyv, 1.0 - lyv)
      wzs = (lzv, 1.0 - lzv)
      ys = (y0 * N_MESH, ((y0 + 1) & (N_MESH - 1)) * N_MESH)
      zs = (z0, (z0 + 1) & (N_MESH - 1))
      rs = ((pk & 3) * PLANE, ((pk >> 2) & 3) * PLANE)
      j = 0
      for dx in range(2):
        for dy in range(2):
          for dz in range(2):
            wj = (wxs[dx] * wys[dy]) * wzs[dz]
            idx_buf[pl.ds(j * 16, 16)] = rs[dx] + ys[dy] + zs[dz]
            for a in range(16):
              stage[j * 16 + a] = emb_t[a] * _bcast_lane(wj, a)
            j += 1
      pltpu.sync_copy(stage, window.at[idx_buf], add=True)
      return g + 1

    lax.while_loop(lambda g: jnp.any((g * 16 + lane) < cnt),
                   dep_group, jnp.int32(0))
    plsc.subcore_barrier()

    pltpu.sync_copy(window.at[pl.ds(s * TSH, TSH)],
                    out_hbm.at[pl.ds(x0 * PLANE + s * TSH, TSH)])
    plsc.subcore_barrier()
    return carry

  lax.fori_loop(0, N_PASS, one_pass, 0)


def _tr_body(in_ref, out_ref):
  out_ref[...] = in_ref[...].T


def _transpose(x):
  return pl.pallas_call(
      _tr_body,
      grid=(N_MESH,),
      in_specs=[pl.BlockSpec((PLANE, N_CH), lambda i: (i, 0))],
      out_specs=pl.BlockSpec((N_CH, PLANE), lambda i: (0, i)),
      out_shape=jax.ShapeDtypeStruct((N_CH, N_MESH ** 3), jnp.float32),
  )(x)


def kernel(positions, embeddings, cell, species):
  box_size = jnp.trace(cell) / 3.0
  spacing = (box_size / N_MESH).astype(jnp.float32)
  sp_v = jnp.full((16,), spacing, jnp.float32)
  pad = N_ATOMS_PAD - positions.shape[0]
  pos = jnp.pad(positions, ((0, pad), (0, 0)))
  emb = jnp.pad(embeddings, ((0, pad), (0, 0)))
  mesh_flat = _deposit(pos[:, 0], pos[:, 1], pos[:, 2], emb, sp_v)
  return _transpose(mesh_flat).reshape(N_CH, N_MESH, N_MESH, N_MESH)
